# baseline (speedup 1.0000x reference)
import jax
import jax.numpy as jnp
from jax.experimental import pallas as pl
from jax.experimental.pallas import tpu as pltpu
from jax.experimental.pallas import tpu_sc as plsc

NUM_EXPERTS = 64
CAPACITY = 80
BLOCK_S = 2048
SC_CHUNK = 512
CHUNKS_PER_BATCH = 8


def _tc_body(x_ref, w_ref, logits_ref, pmax_ref, idx_ref):
    x = x_ref[0]
    w = w_ref[...]
    logits = jax.lax.dot_general(
        x, w, (((1,), (1,)), ((), ())), preferred_element_type=jnp.float32
    )
    logits_ref[0] = logits

    m = jnp.max(logits, axis=-1, keepdims=True)
    ssum = jnp.sum(jnp.exp(logits - m), axis=-1, keepdims=True)
    pmax_ref[0] = 1.0 / ssum

    idx_ref[...] = jnp.argmax(logits, axis=-1).astype(jnp.int32)


def _sc_body(idx_hbm, oh_hbm, idxv, rankv, histv, outv):
    core = jax.lax.axis_index("c")
    sub = jax.lax.axis_index("s")
    b = 2 * core + sub // CHUNKS_PER_BATCH
    cpos = jax.lax.rem(sub, CHUNKS_PER_BATCH)
    nprev = cpos * SC_CHUNK
    tok0 = b * 4096 + nprev

    pltpu.sync_copy(idx_hbm.at[pl.ds(b * 4096, nprev + SC_CHUNK)],
                    idxv.at[pl.ds(0, nprev + SC_CHUNK)])

    zeros16 = jnp.zeros((16,), jnp.int32)
    ones16 = zeros16 + 1
    iota16 = jax.lax.iota(jnp.int32, 16)

    for k in range(NUM_EXPERTS // 16):
        histv[k * 16:(k + 1) * 16] = zeros16

    cal, _ = plsc.scan_count(zeros16)
    adj = 1 - jax.lax.reduce_min(cal, (0,))

    @pl.loop(0, nprev, step=16)
    def _(c):
        e = idxv[pl.ds(c, 16)]
        base = plsc.load_gather(histv, [e])
        dup, last = plsc.scan_count(e)
        plsc.store_scatter(histv, [e], base + dup + adj, mask=last)

    @pl.loop(0, SC_CHUNK, step=16)
    def _(c):
        e = idxv[pl.ds(nprev + c, 16)]
        base = plsc.load_gather(histv, [e])
        dup, last = plsc.scan_count(e)
        rank = base + dup + adj
        rankv[pl.ds(c, 16)] = rank
        plsc.store_scatter(histv, [e], rank, mask=last)

    @pl.loop(0, SC_CHUNK * NUM_EXPERTS, step=16)
    def _(c):
        outv[pl.ds(c, 16)] = zeros16

    @pl.loop(0, SC_CHUNK, step=16)
    def _(c):
        e = idxv[pl.ds(nprev + c, 16)]
        rank = rankv[pl.ds(c, 16)]
        keep = jnp.where(rank <= CAPACITY, ones16, zeros16)
        pos = (zeros16 + c + iota16) * NUM_EXPERTS + e
        plsc.store_scatter(outv, [pos], keep)

    pltpu.sync_copy(outv, oh_hbm.at[pl.ds(tok0 * NUM_EXPERTS,
                                          SC_CHUNK * NUM_EXPERTS)])


def _sc_route(idxp, n_tokens):
    fn = pl.kernel(
        _sc_body,
        out_type=jax.ShapeDtypeStruct((n_tokens * NUM_EXPERTS,), jnp.int32),
        mesh=plsc.VectorSubcoreMesh(core_axis_name="c", subcore_axis_name="s",
                                    num_cores=2, num_subcores=16),
        compiler_params=pltpu.CompilerParams(needs_layout_passes=False),
        scratch_types=[
            pltpu.VMEM((CHUNKS_PER_BATCH * SC_CHUNK,), jnp.int32),
            pltpu.VMEM((SC_CHUNK,), jnp.int32),
            pltpu.VMEM((NUM_EXPERTS,), jnp.int32),
            pltpu.VMEM((SC_CHUNK * NUM_EXPERTS,), jnp.int32),
        ],
    )
    return fn(idxp)


def kernel(hidden_states, W):
    B, S, H = hidden_states.shape
    E = W.shape[0]
    n_sblocks = S // BLOCK_S
    grid = (B, n_sblocks)

    out_shapes = (
        jax.ShapeDtypeStruct((B, S, E), jnp.float32),
        jax.ShapeDtypeStruct((B, S, 1), jnp.float32),
        jax.ShapeDtypeStruct((B * S,), jnp.int32),
    )
    logits, pmax, idxp = pl.pallas_call(
        _tc_body,
        grid=grid,
        in_specs=[
            pl.BlockSpec((1, BLOCK_S, H), lambda b, s: (b, s, 0)),
            pl.BlockSpec((E, H), lambda b, s: (0, 0)),
        ],
        out_specs=[
            pl.BlockSpec((1, BLOCK_S, E), lambda b, s: (b, s, 0)),
            pl.BlockSpec((1, BLOCK_S, 1), lambda b, s: (b, s, 0)),
            pl.BlockSpec((BLOCK_S,), lambda b, s: (b * n_sblocks + s,)),
        ],
        out_shape=out_shapes,
        compiler_params=pltpu.CompilerParams(
            dimension_semantics=("arbitrary", "arbitrary"),
        ),
    )(hidden_states, W)

    oh = _sc_route(idxp, B * S).reshape(B, S, E)
    return (oh, pmax, logits)


# speedup vs baseline: 1.4406x; 1.4406x over previous
import jax
import jax.numpy as jnp
from jax.experimental import pallas as pl
from jax.experimental.pallas import tpu as pltpu

NUM_EXPERTS = 64
CAPACITY = 80
BLOCK_S = 2048
CHUNK = 512


def _router_body(x_ref, w_ref, oh_ref, pmax_ref, logits_ref, counts_ref):
    sb = pl.program_id(1)

    @pl.when(sb == 0)
    def _():
        counts_ref[...] = jnp.zeros_like(counts_ref)

    x = x_ref[0]
    w = w_ref[...]
    logits = jax.lax.dot_general(
        x, w, (((1,), (1,)), ((), ())), preferred_element_type=jnp.float32
    )
    logits_ref[0] = logits

    m = jnp.max(logits, axis=-1, keepdims=True)
    ssum = jnp.sum(jnp.exp(logits - m), axis=-1, keepdims=True)
    pmax_ref[0] = 1.0 / ssum

    idx = jnp.argmax(logits, axis=-1)
    iota = jax.lax.broadcasted_iota(jnp.int32, (BLOCK_S, NUM_EXPERTS), 1)
    oh = (iota == idx[:, None]).astype(jnp.int32)

    r = jax.lax.broadcasted_iota(jnp.int32, (CHUNK, CHUNK), 0)
    c = jax.lax.broadcasted_iota(jnp.int32, (CHUNK, CHUNK), 1)
    tril = (r >= c).astype(jnp.bfloat16)

    counts = counts_ref[...]
    for ci in range(BLOCK_S // CHUNK):
        ohc = oh[ci * CHUNK:(ci + 1) * CHUNK]
        csum = jax.lax.dot_general(
            tril, ohc.astype(jnp.bfloat16), (((1,), (0,)), ((), ())),
            preferred_element_type=jnp.float32,
        ).astype(jnp.int32)
        priority = csum + counts
        keep = (priority <= CAPACITY).astype(jnp.int32)
        oh_ref[0, ci * CHUNK:(ci + 1) * CHUNK, :] = ohc * keep
        counts = counts + csum[CHUNK - 1:CHUNK, :]
    counts_ref[...] = counts


def kernel(hidden_states, W):
    B, S, H = hidden_states.shape
    E = W.shape[0]
    grid = (B, S // BLOCK_S)

    out_shapes = (
        jax.ShapeDtypeStruct((B, S, E), jnp.int32),
        jax.ShapeDtypeStruct((B, S, 1), jnp.float32),
        jax.ShapeDtypeStruct((B, S, E), jnp.float32),
    )
    oh, pmax, logits = pl.pallas_call(
        _router_body,
        grid=grid,
        in_specs=[
            pl.BlockSpec((1, BLOCK_S, H), lambda b, s: (b, s, 0)),
            pl.BlockSpec((E, H), lambda b, s: (0, 0)),
        ],
        out_specs=[
            pl.BlockSpec((1, BLOCK_S, E), lambda b, s: (b, s, 0)),
            pl.BlockSpec((1, BLOCK_S, 1), lambda b, s: (b, s, 0)),
            pl.BlockSpec((1, BLOCK_S, E), lambda b, s: (b, s, 0)),
        ],
        out_shape=out_shapes,
        scratch_shapes=[pltpu.VMEM((1, NUM_EXPERTS), jnp.int32)],
        compiler_params=pltpu.CompilerParams(
            dimension_semantics=("arbitrary", "arbitrary"),
        ),
    )(hidden_states, W)
    return (oh, pmax, logits)
